# baseline (speedup 1.0000x reference)
import jax
import jax.numpy as jnp
from jax.experimental import pallas as pl
from jax.experimental.pallas import tpu as pltpu

_VOCAB = 1000
_EMBED = 256
_HIDDEN = 512
_MAX_SEQ_LEN = 64
_BOS_IDX = 1
_BATCH = 64
_TF_RATIO = 0.5


def _gru_loop_kernel(ytf_ref, emb_ref, W_ih_ref, W_cat_ref,
                     b_ih_ref, b_hh_ref, b_out_ref, out_ref):
    H = _HIDDEN
    iota_v = jax.lax.broadcasted_iota(jnp.int32, (_BATCH, _VOCAB), 1)

    out_ref[0] = jnp.where(iota_v == _BOS_IDX, 0.0, 1e-4).astype(jnp.float32)

    b_ih = b_ih_ref[...]
    b_hh = b_hh_ref[...]
    b_out = b_out_ref[...]
    emb = emb_ref[...]
    W_ih = W_ih_ref[...]
    W_cat = W_cat_ref[...]

    ytf = ytf_ref[...]
    iota_t = jax.lax.broadcasted_iota(jnp.int32, (_BATCH, _MAX_SEQ_LEN), 1)

    h0 = jnp.zeros((_BATCH, H), dtype=jnp.float32)
    tok0 = jnp.full((_BATCH, 1), _BOS_IDX, dtype=jnp.int32)
    gh0 = jnp.zeros((_BATCH, 3 * H), dtype=jnp.float32) + b_hh

    def step(t, carry):
        h, tok, gh = carry
        oh = (tok == iota_v).astype(jnp.float32)
        x = jnp.dot(oh, emb, preferred_element_type=jnp.float32)
        gi = jnp.dot(x, W_ih, preferred_element_type=jnp.float32) + b_ih
        r = jax.nn.sigmoid(gi[:, :H] + gh[:, :H])
        z = jax.nn.sigmoid(gi[:, H:2 * H] + gh[:, H:2 * H])
        n = jnp.tanh(gi[:, 2 * H:] + r * gh[:, 2 * H:])
        h = (1.0 - z) * n + z * h
        p = jnp.dot(h, W_cat, preferred_element_type=jnp.float32)
        gh = p[:, :3 * H] + b_hh
        logits = p[:, 3 * H:] + b_out
        out_ref[t] = logits
        m = jnp.max(logits, axis=1, keepdims=True)
        cand = jnp.where(logits == m, iota_v, _VOCAB)
        greedy = jnp.min(cand, axis=1, keepdims=True).astype(jnp.int32)
        sel = (iota_t == t).astype(jnp.int32)
        y_col = jnp.sum(ytf * sel, axis=1, keepdims=True)
        tok = jnp.where(y_col >= 0, y_col, greedy)
        return h, tok, gh

    jax.lax.fori_loop(1, _MAX_SEQ_LEN, step, (h0, tok0, gh0))


def kernel(y, emb, W_ih, W_hh, b_ih, b_hh, W_out, b_out):
    coin_key = jax.random.key(42)
    cols = [jnp.ones((_BATCH,), jnp.float32)]
    cols += [jax.random.uniform(jax.random.fold_in(coin_key, t), (_BATCH,))
             for t in range(1, _MAX_SEQ_LEN)]
    mask = jnp.stack(cols, axis=1) < _TF_RATIO
    ytf = jnp.where(mask, y.astype(jnp.int32), -1)

    W_cat = jnp.concatenate([W_hh, W_out], axis=1)

    out = pl.pallas_call(
        _gru_loop_kernel,
        out_shape=jax.ShapeDtypeStruct((_MAX_SEQ_LEN, _BATCH, _VOCAB),
                                       jnp.float32),
        compiler_params=pltpu.CompilerParams(
            vmem_limit_bytes=100 * 1024 * 1024),
    )(ytf, emb, W_ih, W_cat,
      b_ih.reshape(1, -1), b_hh.reshape(1, -1), b_out.reshape(1, -1))
    return jnp.swapaxes(out, 0, 1)


# speedup vs baseline: 1.1123x; 1.1123x over previous
import jax
import jax.numpy as jnp
from jax.experimental import pallas as pl
from jax.experimental.pallas import tpu as pltpu

_VOCAB = 1000
_EMBED = 256
_HIDDEN = 512
_MAX_SEQ_LEN = 64
_BOS_IDX = 1
_BATCH = 64
_TF_RATIO = 0.5


def _gru_loop_kernel(ytf_ref, emb_ref, W_ih_ref, W_hh_ref,
                     b_ih_ref, b_hh_ref, W_out_ref, b_out_ref, out_ref):
    H = _HIDDEN
    iota_v = jax.lax.broadcasted_iota(jnp.int32, (_BATCH, _VOCAB), 1)

    out_ref[0] = jnp.where(iota_v == _BOS_IDX, 0.0, 1e-4).astype(jnp.float32)

    b_ih = b_ih_ref[...]
    b_hh = b_hh_ref[...]
    b_out = b_out_ref[...]
    emb = emb_ref[...]
    W_ih = W_ih_ref[...]
    W_hh = W_hh_ref[...]
    W_out = W_out_ref[...]

    ytf = ytf_ref[...]
    iota_t = jax.lax.broadcasted_iota(jnp.int32, (_BATCH, _MAX_SEQ_LEN), 1)

    h = jnp.zeros((_BATCH, H), dtype=jnp.float32)
    tok = jnp.full((_BATCH, 1), _BOS_IDX, dtype=jnp.int32)

    for t in range(1, _MAX_SEQ_LEN):
        oh = (tok == iota_v).astype(jnp.float32)
        x = jnp.dot(oh, emb, preferred_element_type=jnp.float32)
        gi = jnp.dot(x, W_ih, preferred_element_type=jnp.float32) + b_ih
        gh = jnp.dot(h, W_hh, preferred_element_type=jnp.float32) + b_hh
        r = jax.nn.sigmoid(gi[:, :H] + gh[:, :H])
        z = jax.nn.sigmoid(gi[:, H:2 * H] + gh[:, H:2 * H])
        n = jnp.tanh(gi[:, 2 * H:] + r * gh[:, 2 * H:])
        h = (1.0 - z) * n + z * h
        logits = jnp.dot(h, W_out, preferred_element_type=jnp.float32) + b_out
        out_ref[t] = logits
        m = jnp.max(logits, axis=1, keepdims=True)
        cand = jnp.where(logits == m, iota_v, _VOCAB)
        greedy = jnp.min(cand, axis=1, keepdims=True).astype(jnp.int32)
        sel = (iota_t == t).astype(jnp.int32)
        y_col = jnp.sum(ytf * sel, axis=1, keepdims=True)
        tok = jnp.where(y_col >= 0, y_col, greedy)


def kernel(y, emb, W_ih, W_hh, b_ih, b_hh, W_out, b_out):
    coin_key = jax.random.key(42)
    cols = [jnp.ones((_BATCH,), jnp.float32)]
    cols += [jax.random.uniform(jax.random.fold_in(coin_key, t), (_BATCH,))
             for t in range(1, _MAX_SEQ_LEN)]
    mask = jnp.stack(cols, axis=1) < _TF_RATIO
    ytf = jnp.where(mask, y.astype(jnp.int32), -1)

    out = pl.pallas_call(
        _gru_loop_kernel,
        out_shape=jax.ShapeDtypeStruct((_MAX_SEQ_LEN, _BATCH, _VOCAB),
                                       jnp.float32),
        compiler_params=pltpu.CompilerParams(
            vmem_limit_bytes=100 * 1024 * 1024),
    )(ytf, emb, W_ih, W_hh,
      b_ih.reshape(1, -1), b_hh.reshape(1, -1), W_out, b_out.reshape(1, -1))
    return jnp.swapaxes(out, 0, 1)
